# baseline (speedup 1.0000x reference)
import functools

import jax
import jax.numpy as jnp
from jax import lax
from jax.experimental import pallas as pl
from jax.experimental.pallas import tpu as pltpu
from jax.experimental.pallas import tpu_sc as plsc

_UPPER = 0.4
_LOWER = 0.1
_NCLS = 80
_BACKGROUND = _NCLS
_TL = 1000

_NC = 2
_NS = 16
_NW = _NC * _NS
_CHUNK = 128


def _argmax_kernel(cls_ref, idx_ref, logz_ref):
    n = pl.program_id(0)
    t = pl.program_id(1)
    cls = cls_ref[0]
    tl, C = cls.shape
    L = pl.num_programs(1) * tl
    lane_c = lax.broadcasted_iota(jnp.int32, (tl, C), 1)
    rowmax = jnp.max(cls, axis=1, keepdims=True)
    idx = jnp.min(jnp.where(cls == rowmax, lane_c, C), axis=1, keepdims=True)
    logz = rowmax + jnp.log(jnp.sum(jnp.exp(cls - rowmax), axis=1, keepdims=True))
    row = lax.broadcasted_iota(jnp.int32, (tl, 1), 0)
    idx_ref[0] = (n * L + t * tl + row) * C + idx
    logz_ref[0] = logz


def _make_sc_gather(b_pad, b_per_w, width):
    nchunk = b_per_w // _CHUNK
    mesh = plsc.VectorSubcoreMesh(core_axis_name="c", subcore_axis_name="s")

    @functools.partial(
        pl.kernel, mesh=mesh,
        compiler_params=pltpu.CompilerParams(use_tc_tiling_on_sc=False),
        out_type=jax.ShapeDtypeStruct((b_pad // _CHUNK, _CHUNK, width),
                                      jnp.float32),
        scratch_types=[
            pltpu.VMEM((nchunk, _CHUNK), jnp.int32),
            pltpu.VMEM((nchunk, _CHUNK, width), jnp.float32),
            pltpu.SemaphoreType.DMA,
        ],
    )
    def gather_k(table_hbm, idx_hbm, out_hbm, idx_v, rows_v, sem):
        wid = lax.axis_index("s") * _NC + lax.axis_index("c")
        pltpu.sync_copy(idx_hbm.at[pl.ds(wid * nchunk, nchunk)], idx_v)
        handles = [
            pltpu.async_copy(table_hbm.at[idx_v.at[j]], rows_v.at[j], sem)
            for j in range(nchunk)
        ]
        for h in handles:
            h.wait()
        pltpu.sync_copy(rows_v, out_hbm.at[pl.ds(wid * nchunk, nchunk)])

    return gather_k


def _loss_kernel(cls_ref, sel16_ref, base_ref, logz_ref, rois_ref, gtt_ref,
                 gtc_ref, acc_ref):
    t = pl.program_id(1)
    cls = cls_ref[0]
    sel16 = sel16_ref[0]
    base = base_ref[0]
    logz = logz_ref[0]
    rois = rois_ref[0]
    gtt = gtt_ref[0]
    gtc = gtc_ref[0]

    tl, C = cls.shape
    M = gtc.shape[1]

    off = jnp.bitwise_and(base, 3) * 4
    lane16 = lax.broadcasted_iota(jnp.int32, (tl, 16), 1)
    pred = []
    for k in range(4):
        sk = jnp.sum(jnp.where(lane16 == off + k, sel16, 0.0),
                     axis=1, keepdims=True)
        pred.append(rois[:, k:k + 1] + sk)
    px1, py1, px2, py2 = pred

    gx1, gy1, gx2, gy2 = (gtt[k:k + 1, :] for k in range(4))
    area_a = (px2 - px1) * (py2 - py1)
    area_b = (gx2 - gx1) * (gy2 - gy1)
    iw = jnp.maximum(jnp.minimum(px2, gx2) - jnp.maximum(px1, gx1), 0.0)
    ih = jnp.maximum(jnp.minimum(py2, gy2) - jnp.maximum(py1, gy1), 0.0)
    inter = iw * ih
    iou = inter / (area_a + area_b - inter + 1e-9)
    max_iou = jnp.max(iou, axis=1, keepdims=True)
    lane_m = lax.broadcasted_iota(jnp.int32, (tl, M), 1)
    arg = jnp.min(jnp.where(iou == max_iou, lane_m, M), axis=1, keepdims=True)

    pos = max_iou >= _UPPER
    neg = max_iou < _LOWER
    onehot = lane_m == arg
    pos_label = jnp.sum(jnp.where(onehot, gtc, 0.0), axis=1, keepdims=True)
    label = jnp.where(pos, pos_label, float(_BACKGROUND))

    lane_c = lax.broadcasted_iota(jnp.int32, (tl, C), 1)
    logit_at = jnp.sum(
        jnp.where(lane_c.astype(jnp.float32) == label, cls, 0.0),
        axis=1, keepdims=True)
    ce = logz - logit_at
    w = (pos | neg).astype(jnp.float32)

    bl = jnp.zeros((tl, 1), jnp.float32)
    for k in range(4):
        gk = jnp.sum(jnp.where(onehot, gtt[k:k + 1, :], 0.0), axis=1, keepdims=True)
        d = pred[k] - gk
        ad = jnp.abs(d)
        bl = bl + jnp.where(ad < 1.0, 0.5 * d * d, ad - 0.5)
    pw = pos.astype(jnp.float32)

    sums = (jnp.sum(ce * w), jnp.sum(w), jnp.sum(pw),
            jnp.sum(neg.astype(jnp.float32)), jnp.sum(bl * pw))
    lane_o = lax.broadcasted_iota(jnp.int32, (1, 128), 1)
    vec = jnp.zeros((1, 128), jnp.float32)
    for j, sv in enumerate(sums):
        vec = vec + jnp.where(lane_o == j, sv, 0.0)

    @pl.when(t == 0)
    def _init():
        acc_ref[0] = vec

    @pl.when(t != 0)
    def _acc():
        acc_ref[0] = acc_ref[0] + vec


def kernel(rois, cls_scores, bbox_deltas, gt_boxes, gt_clses, device):
    N, L, C = cls_scores.shape
    M = gt_boxes.shape[2]
    T = L // _TL

    base_idx, logz = pl.pallas_call(
        _argmax_kernel,
        grid=(N, T),
        in_specs=[pl.BlockSpec((1, _TL, C), lambda n, t: (n, t, 0))],
        out_specs=[pl.BlockSpec((1, _TL, 1), lambda n, t: (n, t, 0)),
                   pl.BlockSpec((1, _TL, 1), lambda n, t: (n, t, 0))],
        out_shape=[jax.ShapeDtypeStruct((N, L, 1), jnp.int32),
                   jax.ShapeDtypeStruct((N, L, 1), jnp.float32)],
    )(cls_scores)

    B = N * L
    b_per_w = -(-B // (_NW * _CHUNK)) * _CHUNK
    b_pad = b_per_w * _NW
    idx_flat = jax.lax.shift_right_logical(base_idx.reshape(B), 2)
    idx_pad = jnp.pad(idx_flat, (0, b_pad - B)).reshape(b_pad // _CHUNK, _CHUNK)
    table = bbox_deltas.reshape(N * L * C // 4, 16)
    sel16 = _make_sc_gather(b_pad, b_per_w, 16)(table, idx_pad)
    sel16 = sel16.reshape(b_pad, 16)[:B].reshape(N, L, 16)

    gtt = jnp.swapaxes(gt_boxes[:, 0], 1, 2)
    gtc = gt_clses.astype(jnp.float32).reshape(N, 1, M)
    acc = pl.pallas_call(
        _loss_kernel,
        grid=(N, T),
        in_specs=[
            pl.BlockSpec((1, _TL, C), lambda n, t: (n, t, 0)),
            pl.BlockSpec((1, _TL, 16), lambda n, t: (n, t, 0)),
            pl.BlockSpec((1, _TL, 1), lambda n, t: (n, t, 0)),
            pl.BlockSpec((1, _TL, 1), lambda n, t: (n, t, 0)),
            pl.BlockSpec((1, _TL, 4), lambda n, t: (n, t, 0)),
            pl.BlockSpec((1, 4, M), lambda n, t: (n, 0, 0)),
            pl.BlockSpec((1, 1, M), lambda n, t: (n, 0, 0)),
        ],
        out_specs=pl.BlockSpec((1, 1, 128), lambda n, t: (n, 0, 0)),
        out_shape=jax.ShapeDtypeStruct((N, 1, 128), jnp.float32),
    )(cls_scores, sel16, base_idx, logz, rois, gtt, gtc)
    acc = acc[:, 0, :]
    s_ce_w, s_w, s_pos, s_neg, s_bl = (acc[:, j] for j in range(5))
    cls_loss = jnp.sum(s_ce_w / jnp.maximum(s_w, 1.0))
    bbox_loss = jnp.sum(jnp.where(s_pos > 0, s_bl / N, 0.0))
    return (cls_loss, bbox_loss, jnp.sum(s_pos), jnp.sum(s_neg))


# speedup vs baseline: 1.4458x; 1.4458x over previous
import jax
import jax.numpy as jnp
from jax import lax
from jax.experimental import pallas as pl

_UPPER = 0.4
_LOWER = 0.1
_NCLS = 80
_BACKGROUND = _NCLS
_TL = 1000

_HI = jax.lax.Precision.HIGHEST


def _loss_kernel(cls_ref, bd_ref, rois_ref, gtt_ref, gtc_ref, g_ref, acc_ref):
    t = pl.program_id(1)
    cls = cls_ref[0]
    bd = bd_ref[0]
    rois = rois_ref[0]
    gtt = gtt_ref[0]
    gtc = gtc_ref[0]
    g = g_ref[0]

    tl, C = cls.shape
    M = gtc.shape[1]
    D = bd.shape[1]

    lane_c = lax.broadcasted_iota(jnp.int32, (tl, C), 1)
    rowmax = jnp.max(cls, axis=1, keepdims=True)
    idx = jnp.min(jnp.where(cls == rowmax, lane_c, C), axis=1, keepdims=True)

    ones_c = jnp.ones((C, 1), jnp.float32)
    expv = jnp.exp(cls - rowmax)
    logz = rowmax + jnp.log(jnp.dot(expv, ones_c, precision=_HI))

    lane_d = lax.broadcasted_iota(jnp.int32, (tl, D), 1)
    val = jnp.where(lax.shift_right_logical(lane_d, 2) == idx, bd, 0.0)
    s_row = lax.broadcasted_iota(jnp.int32, (D, 4), 0)
    s_col = lax.broadcasted_iota(jnp.int32, (D, 4), 1)
    smat = (jnp.bitwise_and(s_row, 3) == s_col).astype(jnp.float32)
    sel = jnp.dot(val, smat, precision=_HI)
    pred = [rois[:, k:k + 1] + sel[:, k:k + 1] for k in range(4)]
    px1, py1, px2, py2 = pred

    gx1, gy1, gx2, gy2 = (gtt[k:k + 1, :] for k in range(4))
    area_a = (px2 - px1) * (py2 - py1)
    area_b = (gx2 - gx1) * (gy2 - gy1)
    iw = jnp.maximum(jnp.minimum(px2, gx2) - jnp.maximum(px1, gx1), 0.0)
    ih = jnp.maximum(jnp.minimum(py2, gy2) - jnp.maximum(py1, gy1), 0.0)
    inter = iw * ih
    iou = inter / (area_a + area_b - inter + 1e-9)
    max_iou = jnp.max(iou, axis=1, keepdims=True)
    lane_m = lax.broadcasted_iota(jnp.int32, (tl, M), 1)
    arg = jnp.min(jnp.where(iou == max_iou, lane_m, M), axis=1, keepdims=True)

    pos = max_iou >= _UPPER
    neg = max_iou < _LOWER

    onehot = (lane_m == arg).astype(jnp.float32)
    gsel = jnp.dot(onehot, g, precision=_HI)
    pos_label = gsel[:, 4:5]
    label = jnp.where(pos, pos_label, float(_BACKGROUND))

    lab_hit = jnp.where(lane_c.astype(jnp.float32) == label, cls, 0.0)
    logit_at = jnp.dot(lab_hit, ones_c, precision=_HI)
    ce = logz - logit_at
    w = (pos | neg).astype(jnp.float32)

    bl = jnp.zeros((tl, 1), jnp.float32)
    for k in range(4):
        d = pred[k] - gsel[:, k:k + 1]
        ad = jnp.abs(d)
        bl = bl + jnp.where(ad < 1.0, 0.5 * d * d, ad - 0.5)
    pw = pos.astype(jnp.float32)

    sums = (jnp.sum(ce * w), jnp.sum(w), jnp.sum(pw),
            jnp.sum(neg.astype(jnp.float32)), jnp.sum(bl * pw))
    lane_o = lax.broadcasted_iota(jnp.int32, (1, 128), 1)
    vec = jnp.zeros((1, 128), jnp.float32)
    for j, sv in enumerate(sums):
        vec = vec + jnp.where(lane_o == j, sv, 0.0)

    @pl.when(t == 0)
    def _init():
        acc_ref[0] = vec

    @pl.when(t != 0)
    def _acc():
        acc_ref[0] = acc_ref[0] + vec


def kernel(rois, cls_scores, bbox_deltas, gt_boxes, gt_clses, device):
    N, L, C = cls_scores.shape
    M = gt_boxes.shape[2]
    gtt = jnp.swapaxes(gt_boxes[:, 0], 1, 2)
    gtcf = gt_clses.astype(jnp.float32)
    gtc = gtcf.reshape(N, 1, M)
    g = jnp.concatenate(
        [gt_boxes[:, 0], gtcf[:, :, None],
         jnp.zeros((N, M, 3), jnp.float32)], axis=-1)
    T = L // _TL
    acc = pl.pallas_call(
        _loss_kernel,
        grid=(N, T),
        in_specs=[
            pl.BlockSpec((1, _TL, C), lambda n, t: (n, t, 0)),
            pl.BlockSpec((1, _TL, 4 * C), lambda n, t: (n, t, 0)),
            pl.BlockSpec((1, _TL, 4), lambda n, t: (n, t, 0)),
            pl.BlockSpec((1, 4, M), lambda n, t: (n, 0, 0)),
            pl.BlockSpec((1, 1, M), lambda n, t: (n, 0, 0)),
            pl.BlockSpec((1, M, 8), lambda n, t: (n, 0, 0)),
        ],
        out_specs=pl.BlockSpec((1, 1, 128), lambda n, t: (n, 0, 0)),
        out_shape=jax.ShapeDtypeStruct((N, 1, 128), jnp.float32),
    )(cls_scores, bbox_deltas, rois, gtt, gtc, g)
    acc = acc[:, 0, :]
    s_ce_w, s_w, s_pos, s_neg, s_bl = (acc[:, j] for j in range(5))
    cls_loss = jnp.sum(s_ce_w / jnp.maximum(s_w, 1.0))
    bbox_loss = jnp.sum(jnp.where(s_pos > 0, s_bl / N, 0.0))
    return (cls_loss, bbox_loss, jnp.sum(s_pos), jnp.sum(s_neg))


# speedup vs baseline: 2.1682x; 1.4996x over previous
import jax
import jax.numpy as jnp
from jax.experimental import pallas as pl

_NEGATIVE = -2
_UPPER = 0.4
_LOWER = 0.1
_NCLS = 80
_BACKGROUND = _NCLS
_TL = 1000


def _loss_kernel(cls_ref, bd_ref, rois_ref, gtt_ref, gtc_ref, acc_ref):
    t = pl.program_id(1)
    cls = cls_ref[0]
    bd = bd_ref[0]
    rois = rois_ref[0]
    gtt = gtt_ref[0]
    gtc = gtc_ref[0]

    tl, C = cls.shape
    M = gtc.shape[1]

    lane_c = jax.lax.broadcasted_iota(jnp.int32, (tl, C), 1)
    rowmax = jnp.max(cls, axis=1, keepdims=True)
    idx = jnp.min(jnp.where(cls == rowmax, lane_c, C), axis=1, keepdims=True)

    logz = rowmax + jnp.log(jnp.sum(jnp.exp(cls - rowmax), axis=1, keepdims=True))

    D = bd.shape[1]
    lane_d = jax.lax.broadcasted_iota(jnp.int32, (tl, D), 1)
    cls_hit = jax.lax.shift_right_logical(lane_d, 2) == idx
    q = jnp.where(cls_hit, bd, 0.0)
    tail = jnp.concatenate(
        [q[:, 256:D], jnp.zeros((tl, 384 - D), jnp.float32)], axis=1)
    qf = q[:, 0:128] + q[:, 128:256] + tail
    lane_f = jax.lax.broadcasted_iota(jnp.int32, (tl, 128), 1)
    sub = jnp.bitwise_and(lane_f, 3)
    pred = []
    for k in range(4):
        sk = jnp.sum(jnp.where(sub == k, qf, 0.0), axis=1, keepdims=True)
        pred.append(rois[:, k:k + 1] + sk)
    px1, py1, px2, py2 = pred

    gx1, gy1, gx2, gy2 = (gtt[k:k + 1, :] for k in range(4))
    area_a = (px2 - px1) * (py2 - py1)
    area_b = (gx2 - gx1) * (gy2 - gy1)
    iw = jnp.maximum(jnp.minimum(px2, gx2) - jnp.maximum(px1, gx1), 0.0)
    ih = jnp.maximum(jnp.minimum(py2, gy2) - jnp.maximum(py1, gy1), 0.0)
    inter = iw * ih
    iou = inter / (area_a + area_b - inter + 1e-9)
    max_iou = jnp.max(iou, axis=1, keepdims=True)
    lane_m = jax.lax.broadcasted_iota(jnp.int32, (tl, M), 1)
    arg = jnp.min(jnp.where(iou == max_iou, lane_m, M), axis=1, keepdims=True)

    pos = max_iou >= _UPPER
    neg = max_iou < _LOWER
    onehot = lane_m == arg
    pos_label = jnp.sum(jnp.where(onehot, gtc, 0.0), axis=1, keepdims=True)
    label = jnp.where(pos, pos_label, float(_BACKGROUND))

    logit_at = jnp.sum(
        jnp.where(lane_c.astype(jnp.float32) == label, cls, 0.0),
        axis=1, keepdims=True)
    ce = logz - logit_at
    w = (pos | neg).astype(jnp.float32)

    bl = jnp.zeros((tl, 1), jnp.float32)
    for k in range(4):
        gk = jnp.sum(jnp.where(onehot, gtt[k:k + 1, :], 0.0), axis=1, keepdims=True)
        d = pred[k] - gk
        ad = jnp.abs(d)
        bl = bl + jnp.where(ad < 1.0, 0.5 * d * d, ad - 0.5)
    pw = pos.astype(jnp.float32)

    sums = (jnp.sum(ce * w), jnp.sum(w), jnp.sum(pw),
            jnp.sum(neg.astype(jnp.float32)), jnp.sum(bl * pw))
    lane_o = jax.lax.broadcasted_iota(jnp.int32, (1, 128), 1)
    vec = jnp.zeros((1, 128), jnp.float32)
    for j, sv in enumerate(sums):
        vec = vec + jnp.where(lane_o == j, sv, 0.0)

    @pl.when(t == 0)
    def _init():
        acc_ref[0] = vec

    @pl.when(t != 0)
    def _acc():
        acc_ref[0] = acc_ref[0] + vec


def kernel(rois, cls_scores, bbox_deltas, gt_boxes, gt_clses, device):
    N, L, C = cls_scores.shape
    M = gt_boxes.shape[2]
    gtt = jnp.swapaxes(gt_boxes[:, 0], 1, 2)
    gtc = gt_clses.astype(jnp.float32).reshape(N, 1, M)
    T = L // _TL
    acc = pl.pallas_call(
        _loss_kernel,
        grid=(N, T),
        in_specs=[
            pl.BlockSpec((1, _TL, C), lambda n, t: (n, t, 0)),
            pl.BlockSpec((1, _TL, 4 * C), lambda n, t: (n, t, 0)),
            pl.BlockSpec((1, _TL, 4), lambda n, t: (n, t, 0)),
            pl.BlockSpec((1, 4, M), lambda n, t: (n, 0, 0)),
            pl.BlockSpec((1, 1, M), lambda n, t: (n, 0, 0)),
        ],
        out_specs=pl.BlockSpec((1, 1, 128), lambda n, t: (n, 0, 0)),
        out_shape=jax.ShapeDtypeStruct((N, 1, 128), jnp.float32),
    )(cls_scores, bbox_deltas, rois, gtt, gtc)
    acc = acc[:, 0, :]
    s_ce_w, s_w, s_pos, s_neg, s_bl = (acc[:, j] for j in range(5))
    cls_loss = jnp.sum(s_ce_w / jnp.maximum(s_w, 1.0))
    bbox_loss = jnp.sum(jnp.where(s_pos > 0, s_bl / N, 0.0))
    return (cls_loss, bbox_loss, jnp.sum(s_pos), jnp.sum(s_neg))
